# Initial kernel scaffold; baseline (speedup 1.0000x reference)
import jax
import jax.numpy as jnp
from jax.experimental import pallas as pl


def kernel(node_feat, neighbor_feats_l1, neighbor_feats_l2, W_self1, b_self1, W_nbr1, b_nbr1, g1, be1, W_self2, b_self2, W_nbr2, b_nbr2, g2, be2):
    raise NotImplementedError("write your pallas kernel here")


# speedup vs baseline: 1.2055x; 1.2055x over previous
import jax
import jax.numpy as jnp
from jax.experimental import pallas as pl
from jax.experimental.pallas import tpu as pltpu

FEATURE_DIM = 128
HIDDEN_DIM = 256
EMBED_DIM = 256
N_NODES = 10000
N_NBR = 160000

_R_CHUNK = 4000
_N_RSTEPS = N_NBR // _R_CHUNK
_ROW_TILE = 1000
_N_DSTEPS = N_NODES // _ROW_TILE


def _reduce_body(l1_ref, l2_ref, wn1_ref, bn1_ref, wn2_ref, bn2_ref,
                 nbr1_ref, nbr2_ref, acc1_ref, acc2_ref):
    i = pl.program_id(0)

    @pl.when(i == 0)
    def _init():
        acc1_ref[...] = jnp.zeros_like(acc1_ref)
        acc2_ref[...] = jnp.zeros_like(acc2_ref)

    c1 = l1_ref[...].reshape(_R_CHUNK // 8, 8, FEATURE_DIM).sum(axis=0)
    c2 = l2_ref[...].reshape(_R_CHUNK // 8, 8, HIDDEN_DIM).sum(axis=0)
    acc1_ref[...] += c1
    acc2_ref[...] += c2

    @pl.when(i == _N_RSTEPS - 1)
    def _finalize():
        agg1 = acc1_ref[...].sum(axis=0, keepdims=True) * (1.0 / N_NBR)
        agg2 = acc2_ref[...].sum(axis=0, keepdims=True) * (1.0 / N_NBR)
        nbr1_ref[...] = jnp.dot(agg1, wn1_ref[...],
                                preferred_element_type=jnp.float32) + bn1_ref[...]
        nbr2_ref[...] = jnp.dot(agg2, wn2_ref[...],
                                preferred_element_type=jnp.float32) + bn2_ref[...]


def _dense_body(nf_ref, ws1_ref, bs1_ref, nbr1_ref, g1_ref, be1_ref,
                ws2_ref, bs2_ref, nbr2_ref, g2_ref, be2_ref,
                out_ref, ssq_ref):
    i = pl.program_id(0)

    @pl.when(i == 0)
    def _init():
        ssq_ref[...] = jnp.zeros_like(ssq_ref)

    @pl.when(i < _N_DSTEPS)
    def _compute():
        x = nf_ref[...]
        out1 = (jnp.dot(x, ws1_ref[...], preferred_element_type=jnp.float32)
                + bs1_ref[...] + nbr1_ref[...])
        mu1 = jnp.mean(out1, axis=-1, keepdims=True)
        d1 = out1 - mu1
        var1 = jnp.mean(d1 * d1, axis=-1, keepdims=True)
        h1 = jnp.maximum(
            d1 * jax.lax.rsqrt(var1 + 1e-5) * g1_ref[...] + be1_ref[...], 0.0)
        out2 = (jnp.dot(h1, ws2_ref[...], preferred_element_type=jnp.float32)
                + bs2_ref[...] + nbr2_ref[...])
        mu2 = jnp.mean(out2, axis=-1, keepdims=True)
        d2 = out2 - mu2
        var2 = jnp.mean(d2 * d2, axis=-1, keepdims=True)
        h2 = jnp.maximum(
            d2 * jax.lax.rsqrt(var2 + 1e-5) * g2_ref[...] + be2_ref[...], 0.0)
        out_ref[pl.ds(i * _ROW_TILE, _ROW_TILE), :] = h2
        ssq_ref[...] += jnp.sum(h2 * h2, axis=0, keepdims=True)

    @pl.when(i == _N_DSTEPS)
    def _normalize():
        norm = jnp.sqrt(ssq_ref[...])
        scale = 1.0 / jnp.maximum(norm, 1e-12)
        out_ref[...] = out_ref[...] * scale


def kernel(node_feat, neighbor_feats_l1, neighbor_feats_l2, W_self1, b_self1,
           W_nbr1, b_nbr1, g1, be1, W_self2, b_self2, W_nbr2, b_nbr2, g2, be2):
    f32 = jnp.float32
    row = lambda v: v.reshape(1, -1)

    nbr1, nbr2 = pl.pallas_call(
        _reduce_body,
        grid=(_N_RSTEPS,),
        in_specs=[
            pl.BlockSpec((_R_CHUNK, FEATURE_DIM), lambda i: (i, 0)),
            pl.BlockSpec((_R_CHUNK, HIDDEN_DIM), lambda i: (i, 0)),
            pl.BlockSpec((FEATURE_DIM, HIDDEN_DIM), lambda i: (0, 0)),
            pl.BlockSpec((1, HIDDEN_DIM), lambda i: (0, 0)),
            pl.BlockSpec((HIDDEN_DIM, EMBED_DIM), lambda i: (0, 0)),
            pl.BlockSpec((1, EMBED_DIM), lambda i: (0, 0)),
        ],
        out_specs=[
            pl.BlockSpec((1, HIDDEN_DIM), lambda i: (0, 0)),
            pl.BlockSpec((1, EMBED_DIM), lambda i: (0, 0)),
        ],
        out_shape=[
            jax.ShapeDtypeStruct((1, HIDDEN_DIM), f32),
            jax.ShapeDtypeStruct((1, EMBED_DIM), f32),
        ],
        scratch_shapes=[
            pltpu.VMEM((8, FEATURE_DIM), f32),
            pltpu.VMEM((8, HIDDEN_DIM), f32),
        ],
    )(neighbor_feats_l1, neighbor_feats_l2, W_nbr1, row(b_nbr1),
      W_nbr2, row(b_nbr2))

    h2 = pl.pallas_call(
        _dense_body,
        grid=(_N_DSTEPS + 1,),
        in_specs=[
            pl.BlockSpec((_ROW_TILE, FEATURE_DIM),
                         lambda i: (jnp.minimum(i, _N_DSTEPS - 1), 0)),
            pl.BlockSpec((FEATURE_DIM, HIDDEN_DIM), lambda i: (0, 0)),
            pl.BlockSpec((1, HIDDEN_DIM), lambda i: (0, 0)),
            pl.BlockSpec((1, HIDDEN_DIM), lambda i: (0, 0)),
            pl.BlockSpec((1, HIDDEN_DIM), lambda i: (0, 0)),
            pl.BlockSpec((1, HIDDEN_DIM), lambda i: (0, 0)),
            pl.BlockSpec((HIDDEN_DIM, EMBED_DIM), lambda i: (0, 0)),
            pl.BlockSpec((1, EMBED_DIM), lambda i: (0, 0)),
            pl.BlockSpec((1, EMBED_DIM), lambda i: (0, 0)),
            pl.BlockSpec((1, EMBED_DIM), lambda i: (0, 0)),
            pl.BlockSpec((1, EMBED_DIM), lambda i: (0, 0)),
        ],
        out_specs=pl.BlockSpec((N_NODES, EMBED_DIM), lambda i: (0, 0)),
        out_shape=jax.ShapeDtypeStruct((N_NODES, EMBED_DIM), f32),
        scratch_shapes=[pltpu.VMEM((1, EMBED_DIM), f32)],
    )(node_feat, W_self1, row(b_self1), nbr1, row(g1), row(be1),
      W_self2, row(b_self2), nbr2, row(g2), row(be2))

    return h2
